# baseline (speedup 1.0000x reference)
import jax
import jax.numpy as jnp
from jax.experimental import pallas as pl
from jax.experimental.pallas import tpu as pltpu

_K = 3.0
_SLAB = 16
_NEG = -1e9


def _way_score(snm, qb, tail_slab, rq):
    sim = jax.lax.dot_general(
        snm, qb,
        dimension_numbers=(((0,), (0,)), ((), ())),
        preferred_element_type=jnp.float32,
    )

    n_s, n_q = sim.shape
    simb = sim.astype(jnp.bfloat16)
    sim3 = simb.reshape(n_s // _SLAB, _SLAB, n_q)
    a1 = jnp.full((_SLAB, n_q), _NEG, dtype=jnp.bfloat16)
    a2 = a1
    a3 = a1
    n3 = sim3.shape[0]
    n_t = tail_slab.shape[0] // _SLAB
    for i in range(n3 + n_t):
        if i < n3:
            v = sim3[i]
        else:
            j = i - n3
            v = tail_slab[j * _SLAB:(j + 1) * _SLAB]
        t1 = jnp.maximum(a1, v)
        d1 = jnp.minimum(a1, v)
        t2 = jnp.maximum(a2, d1)
        d2 = jnp.minimum(a2, d1)
        t3 = jnp.maximum(a3, d2)
        a1, a2, a3 = t1, t2, t3

    neg = jnp.array(_NEG, dtype=jnp.bfloat16)
    zero = jnp.array(0.0, dtype=jnp.bfloat16)
    one = jnp.array(1.0, dtype=jnp.bfloat16)
    cand = jnp.concatenate([a1, a2, a3], axis=0)
    m1 = jnp.max(cand, axis=0, keepdims=True)
    lt1 = cand < m1
    c1 = jnp.sum(jnp.where(lt1, zero, one), axis=0, keepdims=True)
    x2 = jnp.where(lt1, cand, neg)
    m2 = jnp.max(x2, axis=0, keepdims=True)
    lt2 = x2 < m2
    c2 = jnp.sum(jnp.where(lt2, zero, one), axis=0, keepdims=True)
    x3 = jnp.where(lt2, x2, neg)
    m3 = jnp.max(x3, axis=0, keepdims=True)

    c1 = c1.astype(jnp.float32)
    c2 = c2.astype(jnp.float32)
    b1 = jnp.minimum(c1, _K)
    b2 = jnp.minimum(_K - b1, c2)
    b3 = _K - b1 - b2
    top3 = (m1.astype(jnp.float32) * b1 + m2.astype(jnp.float32) * b2
            + m3.astype(jnp.float32) * b3)

    return jnp.sum(top3 * rq)


def _knn_step(q_ref, s_ref, out_ref):
    n_b = q_ref.shape[0]
    n_way = s_ref.shape[1]
    lane = jax.lax.broadcasted_iota(jnp.int32, (8, 128), 1)
    for bb in range(n_b):
        q = q_ref[bb]
        rq = jax.lax.rsqrt(jnp.maximum(jnp.sum(q * q, axis=0, keepdims=True), 1e-24))
        qb = q.astype(jnp.bfloat16)

        sns = []
        for c in range(n_way):
            s = s_ref[bb, c]
            r = jax.lax.rsqrt(
                jnp.maximum(jnp.sum(s * s, axis=0, keepdims=True), 1e-24))
            sns.append(s.astype(jnp.bfloat16) * r.astype(jnp.bfloat16))

        n_s = sns[0].shape[1]
        main = (n_s // 256) * 256
        tails = jnp.concatenate([sn[:, main:] for sn in sns], axis=1)
        sim_tail = jax.lax.dot_general(
            tails, qb,
            dimension_numbers=(((0,), (0,)), ((), ())),
            preferred_element_type=jnp.float32,
        ).astype(jnp.bfloat16)

        t_w = sim_tail.shape[0] // n_way
        row = jnp.zeros((8, 128), jnp.float32)
        for c in range(n_way):
            tail_slab = sim_tail[c * t_w:(c + 1) * t_w]
            score = _way_score(sns[c][:, :main], qb, tail_slab, rq)
            row = jnp.where(lane == c, score, row)
        out_ref[bb] = row


def kernel(query_features, support_features):
    B, D, h, w = query_features.shape
    Way = support_features.shape[1]
    hw = h * w
    q = query_features.reshape(B, D, hw)

    bs = 8
    scores = pl.pallas_call(
        _knn_step,
        grid=(B // bs,),
        in_specs=[
            pl.BlockSpec((bs, D, hw), lambda b: (b, 0, 0)),
            pl.BlockSpec((bs, Way, D, hw), lambda b: (b, 0, 0, 0)),
        ],
        out_specs=pl.BlockSpec((bs, 8, 128), lambda b: (b, 0, 0)),
        out_shape=jax.ShapeDtypeStruct((B, 8, 128), jnp.float32),
    )(q, support_features)
    return scores[:, 0, :Way]


# speedup vs baseline: 1.0180x; 1.0180x over previous
import jax
import jax.numpy as jnp
from jax.experimental import pallas as pl
from jax.experimental.pallas import tpu as pltpu

_K = 3.0
_SLAB = 16
_NEG = -1e9


def _way_score(snm, qb, tail_slab, rq):
    sim = jax.lax.dot_general(
        snm, qb,
        dimension_numbers=(((0,), (0,)), ((), ())),
        preferred_element_type=jnp.float32,
    )

    n_s, n_q = sim.shape
    simb = sim.astype(jnp.bfloat16)
    sim3 = simb.reshape(n_s // _SLAB, _SLAB, n_q)
    a1 = jnp.full((_SLAB, n_q), _NEG, dtype=jnp.bfloat16)
    a2 = a1
    a3 = a1
    n3 = sim3.shape[0]
    n_t = tail_slab.shape[0] // _SLAB
    for i in range(n3 + n_t):
        if i < n3:
            v = sim3[i]
        else:
            j = i - n3
            v = tail_slab[j * _SLAB:(j + 1) * _SLAB]
        t1 = jnp.maximum(a1, v)
        d1 = jnp.minimum(a1, v)
        t2 = jnp.maximum(a2, d1)
        d2 = jnp.minimum(a2, d1)
        t3 = jnp.maximum(a3, d2)
        a1, a2, a3 = t1, t2, t3

    neg = jnp.array(_NEG, dtype=jnp.bfloat16)
    zero = jnp.array(0.0, dtype=jnp.bfloat16)
    one = jnp.array(1.0, dtype=jnp.bfloat16)
    cand = jnp.concatenate([a1, a2, a3], axis=0)
    m1 = jnp.max(cand, axis=0, keepdims=True)
    lt1 = cand < m1
    c1 = jnp.sum(jnp.where(lt1, zero, one), axis=0, keepdims=True)
    x2 = jnp.where(lt1, cand, neg)
    m2 = jnp.max(x2, axis=0, keepdims=True)
    lt2 = x2 < m2
    c2 = jnp.sum(jnp.where(lt2, zero, one), axis=0, keepdims=True)
    x3 = jnp.where(lt2, x2, neg)
    m3 = jnp.max(x3, axis=0, keepdims=True)

    c1 = c1.astype(jnp.float32)
    c2 = c2.astype(jnp.float32)
    b1 = jnp.minimum(c1, _K)
    b2 = jnp.minimum(_K - b1, c2)
    b3 = _K - b1 - b2
    top3 = (m1.astype(jnp.float32) * b1 + m2.astype(jnp.float32) * b2
            + m3.astype(jnp.float32) * b3)

    return jnp.sum(top3 * rq)


def _knn_step(q_ref, s_ref, out_ref):
    n_b = q_ref.shape[0]
    n_way = s_ref.shape[1]
    lane = jax.lax.broadcasted_iota(jnp.int32, (8, 128), 1)
    for bb in range(n_b):
        q = q_ref[bb]
        rq = jax.lax.rsqrt(jnp.maximum(jnp.sum(q * q, axis=0, keepdims=True), 1e-24))
        qb = q.astype(jnp.bfloat16)

        sns = []
        for c in range(n_way):
            s = s_ref[bb, c]
            r = jax.lax.rsqrt(
                jnp.maximum(jnp.sum(s * s, axis=0, keepdims=True), 1e-24))
            sns.append(s.astype(jnp.bfloat16) * r.astype(jnp.bfloat16))

        n_s = sns[0].shape[1]
        main = (n_s // 256) * 256
        tails = jnp.concatenate([sn[:, main:] for sn in sns], axis=1)
        sim_tail = jax.lax.dot_general(
            tails, qb,
            dimension_numbers=(((0,), (0,)), ((), ())),
            preferred_element_type=jnp.float32,
        ).astype(jnp.bfloat16)

        t_w = sim_tail.shape[0] // n_way
        row = jnp.zeros((8, 128), jnp.float32)
        for c in range(n_way):
            tail_slab = sim_tail[c * t_w:(c + 1) * t_w]
            score = _way_score(sns[c][:, :main], qb, tail_slab, rq)
            row = jnp.where(lane == c, score, row)
        out_ref[bb] = row


def kernel(query_features, support_features):
    B, D, h, w = query_features.shape
    Way = support_features.shape[1]
    hw = h * w
    q = query_features.reshape(B, D, hw)

    bs = 4
    scores = pl.pallas_call(
        _knn_step,
        grid=(B // bs,),
        in_specs=[
            pl.BlockSpec((bs, D, hw), lambda b: (b, 0, 0)),
            pl.BlockSpec((bs, Way, D, hw), lambda b: (b, 0, 0, 0)),
        ],
        out_specs=pl.BlockSpec((bs, 8, 128), lambda b: (b, 0, 0)),
        out_shape=jax.ShapeDtypeStruct((B, 8, 128), jnp.float32),
    )(q, support_features)
    return scores[:, 0, :Way]
